# Initial kernel scaffold; baseline (speedup 1.0000x reference)
import jax
import jax.numpy as jnp
from jax.experimental import pallas as pl


def kernel(x, batch, emb_0, emb_1, emb_2, emb_3, emb_4, emb_5, emb_6, emb_7, emb_8):
    raise NotImplementedError("write your pallas kernel here")


# speedup vs baseline: 14.0203x; 14.0203x over previous
import functools

import jax
import jax.numpy as jnp
from jax.experimental import pallas as pl
from jax.experimental.pallas import tpu as pltpu

_FEATURE_DIMS = [119, 5, 12, 12, 10, 6, 6, 2, 2]
_IN_CHANNELS = 256
_K = sum(_FEATURE_DIMS)
_K_PAD = 184
_BLOCK = 2000


def _body(x_ref, tab_ref, out_ref):
    xb = x_ref[...]
    iota = jax.lax.broadcasted_iota(jnp.int32, (_BLOCK, _K_PAD), 1)
    oh = jnp.zeros((_BLOCK, _K_PAD), jnp.float32)
    off = 0
    for i, d in enumerate(_FEATURE_DIMS):
        col = jax.lax.broadcast_in_dim(xb[:, i] + off, (_BLOCK, _K_PAD), (0,))
        oh = oh + (iota == col).astype(jnp.float32)
        off += d
    out_ref[...] = jnp.dot(oh, tab_ref[...], preferred_element_type=jnp.float32)


def kernel(x, batch, emb_0, emb_1, emb_2, emb_3, emb_4, emb_5, emb_6, emb_7,
           emb_8):
    del batch
    embs = [emb_0, emb_1, emb_2, emb_3, emb_4, emb_5, emb_6, emb_7, emb_8]
    table = jnp.concatenate(embs, axis=0)
    table = jnp.pad(table, ((0, _K_PAD - _K), (0, 0)))
    n = x.shape[0]
    grid = n // _BLOCK
    return pl.pallas_call(
        _body,
        grid=(grid,),
        in_specs=[
            pl.BlockSpec((_BLOCK, len(_FEATURE_DIMS)), lambda i: (i, 0)),
            pl.BlockSpec((_K_PAD, _IN_CHANNELS), lambda i: (0, 0)),
        ],
        out_specs=pl.BlockSpec((_BLOCK, _IN_CHANNELS), lambda i: (i, 0)),
        out_shape=jax.ShapeDtypeStruct((n, _IN_CHANNELS), jnp.float32),
        compiler_params=pltpu.CompilerParams(
            dimension_semantics=("arbitrary",),
        ),
    )(x.astype(jnp.int32), table)


# speedup vs baseline: 16.1223x; 1.1499x over previous
import numpy as np

import jax
import jax.numpy as jnp
from jax.experimental import pallas as pl
from jax.experimental.pallas import tpu as pltpu

_FEATURE_DIMS = [119, 5, 12, 12, 10, 6, 6, 2, 2]
_NF = len(_FEATURE_DIMS)
_NF_PAD = 16
_IN_CHANNELS = 256
_K = sum(_FEATURE_DIMS)
_K_PAD = 184
_BLOCK = 2000


def _body(x_ref, m_ref, il_ref, thi_ref, tlo_ref, out_ref):
    xb = x_ref[...].astype(jnp.bfloat16)
    v = jnp.dot(xb, m_ref[...], preferred_element_type=jnp.float32)
    il = jax.lax.broadcast_in_dim(il_ref[0, :], (_BLOCK, _K_PAD), (1,))
    oh = (v == il).astype(jnp.bfloat16)
    out_ref[...] = (
        jnp.dot(oh, thi_ref[...], preferred_element_type=jnp.float32)
        + jnp.dot(oh, tlo_ref[...], preferred_element_type=jnp.float32)
    )


def kernel(x, batch, emb_0, emb_1, emb_2, emb_3, emb_4, emb_5, emb_6, emb_7,
           emb_8):
    del batch
    embs = [emb_0, emb_1, emb_2, emb_3, emb_4, emb_5, emb_6, emb_7, emb_8]
    table = jnp.concatenate(embs, axis=0)
    table = jnp.pad(table, ((0, _K_PAD - _K), (0, 0)))
    t_hi = table.astype(jnp.bfloat16)
    t_lo = (table - t_hi.astype(jnp.float32)).astype(jnp.bfloat16)

    offs = np.zeros(_NF, np.int32)
    feat_of_col = np.full(_K_PAD, _NF, np.int32)
    acc = 0
    for i, d in enumerate(_FEATURE_DIMS):
        offs[i] = acc
        feat_of_col[acc:acc + d] = i
        acc += d
    m_sel = np.zeros((_NF_PAD, _K_PAD), np.float32)
    for r in range(_K):
        m_sel[feat_of_col[r], r] = 1.0
    il = np.full((8, _K_PAD), -1.0, np.float32)
    for r in range(_K):
        il[:, r] = float(r - offs[feat_of_col[r]])

    n = x.shape[0]
    x_pad = jnp.concatenate(
        [x.astype(jnp.int32),
         jnp.zeros((n, _NF_PAD - _NF), jnp.int32)], axis=1)

    grid = n // _BLOCK
    return pl.pallas_call(
        _body,
        grid=(grid,),
        in_specs=[
            pl.BlockSpec((_BLOCK, _NF_PAD), lambda i: (i, 0)),
            pl.BlockSpec((_NF_PAD, _K_PAD), lambda i: (0, 0)),
            pl.BlockSpec((8, _K_PAD), lambda i: (0, 0)),
            pl.BlockSpec((_K_PAD, _IN_CHANNELS), lambda i: (0, 0)),
            pl.BlockSpec((_K_PAD, _IN_CHANNELS), lambda i: (0, 0)),
        ],
        out_specs=pl.BlockSpec((_BLOCK, _IN_CHANNELS), lambda i: (i, 0)),
        out_shape=jax.ShapeDtypeStruct((n, _IN_CHANNELS), jnp.float32),
        compiler_params=pltpu.CompilerParams(
            dimension_semantics=("arbitrary",),
        ),
    )(x_pad, jnp.asarray(m_sel, jnp.bfloat16), jnp.asarray(il), t_hi, t_lo)


# speedup vs baseline: 18.3180x; 1.1362x over previous
import numpy as np

import jax
import jax.numpy as jnp
from jax.experimental import pallas as pl
from jax.experimental.pallas import tpu as pltpu

_FEATURE_DIMS = [119, 5, 12, 12, 10, 6, 6, 2, 2]
_NF = len(_FEATURE_DIMS)
_NF_PAD = 16
_IN_CHANNELS = 256
_K = sum(_FEATURE_DIMS)
_K_PAD = 184
_BLOCK = 5000


def _body(x_ref, m_ref, il_ref, thi_ref, tlo_ref, out_ref):
    xb = x_ref[...].astype(jnp.bfloat16)
    v = jnp.dot(xb, m_ref[...], preferred_element_type=jnp.float32)
    il = jax.lax.broadcast_in_dim(il_ref[0, :], (_BLOCK, _K_PAD), (1,))
    oh = (v == il).astype(jnp.bfloat16)
    out_ref[...] = (
        jnp.dot(oh, thi_ref[...], preferred_element_type=jnp.float32)
        + jnp.dot(oh, tlo_ref[...], preferred_element_type=jnp.float32)
    )


def kernel(x, batch, emb_0, emb_1, emb_2, emb_3, emb_4, emb_5, emb_6, emb_7,
           emb_8):
    del batch
    embs = [emb_0, emb_1, emb_2, emb_3, emb_4, emb_5, emb_6, emb_7, emb_8]
    table = jnp.concatenate(embs, axis=0)
    table = jnp.pad(table, ((0, _K_PAD - _K), (0, 0)))
    t_hi = table.astype(jnp.bfloat16)
    t_lo = (table - t_hi.astype(jnp.float32)).astype(jnp.bfloat16)

    offs = np.zeros(_NF, np.int32)
    feat_of_col = np.full(_K_PAD, _NF, np.int32)
    acc = 0
    for i, d in enumerate(_FEATURE_DIMS):
        offs[i] = acc
        feat_of_col[acc:acc + d] = i
        acc += d
    m_sel = np.zeros((_NF_PAD, _K_PAD), np.float32)
    for r in range(_K):
        m_sel[feat_of_col[r], r] = 1.0
    il = np.full((8, _K_PAD), -1.0, np.float32)
    for r in range(_K):
        il[:, r] = float(r - offs[feat_of_col[r]])

    n = x.shape[0]
    x_pad = jnp.concatenate(
        [x.astype(jnp.int32),
         jnp.zeros((n, _NF_PAD - _NF), jnp.int32)], axis=1)

    grid = n // _BLOCK
    return pl.pallas_call(
        _body,
        grid=(grid,),
        in_specs=[
            pl.BlockSpec((_BLOCK, _NF_PAD), lambda i: (i, 0)),
            pl.BlockSpec((_NF_PAD, _K_PAD), lambda i: (0, 0)),
            pl.BlockSpec((8, _K_PAD), lambda i: (0, 0)),
            pl.BlockSpec((_K_PAD, _IN_CHANNELS), lambda i: (0, 0)),
            pl.BlockSpec((_K_PAD, _IN_CHANNELS), lambda i: (0, 0)),
        ],
        out_specs=pl.BlockSpec((_BLOCK, _IN_CHANNELS), lambda i: (i, 0)),
        out_shape=jax.ShapeDtypeStruct((n, _IN_CHANNELS), jnp.float32),
        compiler_params=pltpu.CompilerParams(
            dimension_semantics=("arbitrary",),
        ),
    )(x_pad, jnp.asarray(m_sel, jnp.bfloat16), jnp.asarray(il), t_hi, t_lo)


# speedup vs baseline: 18.9029x; 1.0319x over previous
import numpy as np

import jax
import jax.numpy as jnp
from jax.experimental import pallas as pl
from jax.experimental.pallas import tpu as pltpu

_FEATURE_DIMS = [119, 5, 12, 12, 10, 6, 6, 2, 2]
_NF = len(_FEATURE_DIMS)
_NF_PAD = 16
_IN_CHANNELS = 256
_K = sum(_FEATURE_DIMS)
_K_PAD = 184
_BLOCK = 10000


def _body(x_ref, m_ref, il_ref, thi_ref, tlo_ref, out_ref):
    xb = x_ref[...].astype(jnp.bfloat16)
    v = jnp.dot(xb, m_ref[...], preferred_element_type=jnp.float32)
    il = jax.lax.broadcast_in_dim(il_ref[0, :], (_BLOCK, _K_PAD), (1,))
    oh = (v == il).astype(jnp.bfloat16)
    out_ref[...] = (
        jnp.dot(oh, thi_ref[...], preferred_element_type=jnp.float32)
        + jnp.dot(oh, tlo_ref[...], preferred_element_type=jnp.float32)
    )


def kernel(x, batch, emb_0, emb_1, emb_2, emb_3, emb_4, emb_5, emb_6, emb_7,
           emb_8):
    del batch
    embs = [emb_0, emb_1, emb_2, emb_3, emb_4, emb_5, emb_6, emb_7, emb_8]
    table = jnp.concatenate(embs, axis=0)
    table = jnp.pad(table, ((0, _K_PAD - _K), (0, 0)))
    t_hi = table.astype(jnp.bfloat16)
    t_lo = (table - t_hi.astype(jnp.float32)).astype(jnp.bfloat16)

    offs = np.zeros(_NF, np.int32)
    feat_of_col = np.full(_K_PAD, _NF, np.int32)
    acc = 0
    for i, d in enumerate(_FEATURE_DIMS):
        offs[i] = acc
        feat_of_col[acc:acc + d] = i
        acc += d
    m_sel = np.zeros((_NF_PAD, _K_PAD), np.float32)
    for r in range(_K):
        m_sel[feat_of_col[r], r] = 1.0
    il = np.full((8, _K_PAD), -1.0, np.float32)
    for r in range(_K):
        il[:, r] = float(r - offs[feat_of_col[r]])

    n = x.shape[0]
    x_pad = jnp.concatenate(
        [x.astype(jnp.int32),
         jnp.zeros((n, _NF_PAD - _NF), jnp.int32)], axis=1)

    grid = n // _BLOCK
    return pl.pallas_call(
        _body,
        grid=(grid,),
        in_specs=[
            pl.BlockSpec((_BLOCK, _NF_PAD), lambda i: (i, 0)),
            pl.BlockSpec((_NF_PAD, _K_PAD), lambda i: (0, 0)),
            pl.BlockSpec((8, _K_PAD), lambda i: (0, 0)),
            pl.BlockSpec((_K_PAD, _IN_CHANNELS), lambda i: (0, 0)),
            pl.BlockSpec((_K_PAD, _IN_CHANNELS), lambda i: (0, 0)),
        ],
        out_specs=pl.BlockSpec((_BLOCK, _IN_CHANNELS), lambda i: (i, 0)),
        out_shape=jax.ShapeDtypeStruct((n, _IN_CHANNELS), jnp.float32),
        compiler_params=pltpu.CompilerParams(
            dimension_semantics=("arbitrary",),
        ),
    )(x_pad, jnp.asarray(m_sel, jnp.bfloat16), jnp.asarray(il), t_hi, t_lo)
